# baseline (speedup 1.0000x reference)
import jax
import jax.numpy as jnp
from jax.experimental import pallas as pl
from jax.experimental.pallas import tpu as pltpu


def _copy_body(k_ref, v_ref, ok_ref, ov_ref, sem_k, sem_v):
    ck = pltpu.make_async_copy(k_ref, ok_ref, sem_k)
    cv = pltpu.make_async_copy(v_ref, ov_ref, sem_v)
    ck.start()
    cv.start()
    ck.wait()
    cv.wait()


def kernel(k_new, v_new, k_cache, v_cache):
    del k_cache, v_cache
    shape = jax.ShapeDtypeStruct(k_new.shape, k_new.dtype)
    hbm = pl.BlockSpec(memory_space=pltpu.MemorySpace.HBM)
    out_k, out_v = pl.pallas_call(
        _copy_body,
        in_specs=[hbm, hbm],
        out_specs=[hbm, hbm],
        out_shape=[shape, shape],
        scratch_shapes=[pltpu.SemaphoreType.DMA, pltpu.SemaphoreType.DMA],
    )(k_new, v_new)
    return (out_k, out_v)


# speedup vs baseline: 30.1888x; 30.1888x over previous
import jax
import jax.numpy as jnp
from jax.experimental import pallas as pl
from jax.experimental.pallas import tpu as pltpu


def _copy_body(k_hbm, v_hbm, ok_hbm, ov_hbm, kb, vb, sem_k, sem_v):
    ck = pltpu.make_async_copy(k_hbm, kb, sem_k)
    cv = pltpu.make_async_copy(v_hbm, vb, sem_v)
    ck.start()
    cv.start()
    ck.wait()
    cko = pltpu.make_async_copy(kb, ok_hbm, sem_k)
    cko.start()
    cv.wait()
    cvo = pltpu.make_async_copy(vb, ov_hbm, sem_v)
    cvo.start()
    cko.wait()
    cvo.wait()


def kernel(k_new, v_new, k_cache, v_cache):
    del k_cache, v_cache
    shape = jax.ShapeDtypeStruct(k_new.shape, k_new.dtype)
    hbm = pl.BlockSpec(memory_space=pltpu.MemorySpace.HBM)
    out_k, out_v = pl.pallas_call(
        _copy_body,
        in_specs=[hbm, hbm],
        out_specs=[hbm, hbm],
        out_shape=[shape, shape],
        scratch_shapes=[
            pltpu.VMEM(shape.shape, shape.dtype),
            pltpu.VMEM(shape.shape, shape.dtype),
            pltpu.SemaphoreType.DMA,
            pltpu.SemaphoreType.DMA,
        ],
    )(k_new, v_new)
    return (out_k, out_v)


# speedup vs baseline: 32.1309x; 1.0643x over previous
import jax
import jax.numpy as jnp
from jax.experimental import pallas as pl
from jax.experimental.pallas import tpu as pltpu

_CHUNKS = 4
_ROWS = 32 // _CHUNKS


def _copy_body(k_hbm, v_hbm, ok_hbm, ov_hbm, kb, vb, sem_ik, sem_iv, sem_ok, sem_ov):
    def sl(i):
        return pl.ds(i * _ROWS, _ROWS)

    ins_k = [pltpu.make_async_copy(k_hbm.at[sl(i)], kb.at[sl(i)], sem_ik)
             for i in range(_CHUNKS)]
    ins_v = [pltpu.make_async_copy(v_hbm.at[sl(i)], vb.at[sl(i)], sem_iv)
             for i in range(_CHUNKS)]
    outs_k = [pltpu.make_async_copy(kb.at[sl(i)], ok_hbm.at[sl(i)], sem_ok)
              for i in range(_CHUNKS)]
    outs_v = [pltpu.make_async_copy(vb.at[sl(i)], ov_hbm.at[sl(i)], sem_ov)
              for i in range(_CHUNKS)]
    for i in range(_CHUNKS):
        ins_k[i].start()
        ins_v[i].start()
    for i in range(_CHUNKS):
        ins_k[i].wait()
        outs_k[i].start()
        ins_v[i].wait()
        outs_v[i].start()
    for i in range(_CHUNKS):
        outs_k[i].wait()
        outs_v[i].wait()


def kernel(k_new, v_new, k_cache, v_cache):
    del k_cache, v_cache
    shape = jax.ShapeDtypeStruct(k_new.shape, k_new.dtype)
    hbm = pl.BlockSpec(memory_space=pltpu.MemorySpace.HBM)
    out_k, out_v = pl.pallas_call(
        _copy_body,
        in_specs=[hbm, hbm],
        out_specs=[hbm, hbm],
        out_shape=[shape, shape],
        scratch_shapes=[
            pltpu.VMEM(shape.shape, shape.dtype),
            pltpu.VMEM(shape.shape, shape.dtype),
            pltpu.SemaphoreType.DMA,
            pltpu.SemaphoreType.DMA,
            pltpu.SemaphoreType.DMA,
            pltpu.SemaphoreType.DMA,
        ],
    )(k_new, v_new)
    return (out_k, out_v)
